# baseline (speedup 1.0000x reference)
import jax
import jax.numpy as jnp
from jax.experimental import pallas as pl
from jax.experimental.pallas import tpu as pltpu


def _make_pos_broadcast_kernel(B, H, W, half, n_cores):
    per_core = B // n_cores

    def _pos_broadcast_kernel(row_ref, col_ref, out_ref, scratch, sem):
        core = pl.program_id(0)
        cols_t = col_ref[:W, :].T
        rows_t = row_ref[:H, :].T
        col_block = pltpu.repeat(cols_t, H, axis=1)
        row_block = jnp.repeat(rows_t, W, axis=1)
        for s in range(scratch.shape[0]):
            scratch[s, :half, :] = col_block
            scratch[s, half:, :] = row_block
        for j in range(per_core):
            b = core * per_core + j
            pltpu.make_async_copy(
                scratch.at[j % scratch.shape[0]], out_ref.at[b], sem
            ).start(priority=j % 2)
        for _ in range(per_core):
            pltpu.make_async_copy(scratch.at[0], out_ref.at[0], sem).wait()

    return _pos_broadcast_kernel


def kernel(feature, row_embed, col_embed):
    B, C, H, W = feature.shape
    half = C // 2
    n_cores = 2 if B % 2 == 0 else 1
    out = pl.pallas_call(
        _make_pos_broadcast_kernel(B, H, W, half, n_cores),
        grid=(n_cores,),
        in_specs=[
            pl.BlockSpec((row_embed.shape[0], half), lambda i: (0, 0)),
            pl.BlockSpec((col_embed.shape[0], half), lambda i: (0, 0)),
        ],
        out_specs=pl.BlockSpec(memory_space=pl.ANY),
        out_shape=jax.ShapeDtypeStruct((B, C, H * W), row_embed.dtype),
        scratch_shapes=[
            pltpu.VMEM((2, C, H * W), row_embed.dtype),
            pltpu.SemaphoreType.DMA,
        ],
        compiler_params=pltpu.CompilerParams(
            dimension_semantics=("parallel",),
        ),
    )(row_embed, col_embed)
    return out.reshape(B, C, H, W)


# speedup vs baseline: 1.0573x; 1.0573x over previous
import jax
import jax.numpy as jnp
from jax.experimental import pallas as pl
from jax.experimental.pallas import tpu as pltpu


def _make_pos_broadcast_kernel(H, W, half):
    def _pos_broadcast_kernel(row_ref, col_ref, out_ref, scratch):
        @pl.when(pl.program_id(0) == 0)
        def _build():
            cols_t = col_ref[:W, :].T
            rows_t = row_ref[:H, :].T
            scratch[:half, :] = pltpu.repeat(cols_t, H, axis=1)
            scratch[half:, :] = jnp.repeat(rows_t, W, axis=1)

        out_ref[0] = scratch[...]

    return _pos_broadcast_kernel


def kernel(feature, row_embed, col_embed):
    B, C, H, W = feature.shape
    half = C // 2
    out = pl.pallas_call(
        _make_pos_broadcast_kernel(H, W, half),
        grid=(B,),
        in_specs=[
            pl.BlockSpec((row_embed.shape[0], half), lambda b: (0, 0)),
            pl.BlockSpec((col_embed.shape[0], half), lambda b: (0, 0)),
        ],
        out_specs=pl.BlockSpec((1, C, H * W), lambda b: (b, 0, 0)),
        out_shape=jax.ShapeDtypeStruct((B, C, H * W), row_embed.dtype),
        scratch_shapes=[
            pltpu.VMEM((C, H * W), row_embed.dtype),
        ],
    )(row_embed, col_embed)
    return out.reshape(B, C, H, W)


# speedup vs baseline: 1.0607x; 1.0032x over previous
import jax
import jax.numpy as jnp
from jax.experimental import pallas as pl
from jax.experimental.pallas import tpu as pltpu

_LANE_CHUNKS = 2


def _make_pos_broadcast_kernel(B, H, W, half):
    def _pos_broadcast_kernel(row_ref, col_ref, out_ref, scratch, sem):
        cols_t = col_ref[:W, :].T
        rows_t = row_ref[:H, :].T
        scratch[:half, :] = pltpu.repeat(cols_t, H, axis=1)
        scratch[half:, :] = jnp.repeat(rows_t, W, axis=1)
        hw = H * W
        chunk = hw // _LANE_CHUNKS
        for b in range(B):
            for k in range(_LANE_CHUNKS):
                sl = pl.ds(k * chunk, chunk)
                pltpu.make_async_copy(
                    scratch.at[:, sl], out_ref.at[b, :, sl], sem
                ).start()
        for _ in range(B * _LANE_CHUNKS):
            pltpu.make_async_copy(
                scratch.at[:, pl.ds(0, chunk)],
                out_ref.at[0, :, pl.ds(0, chunk)],
                sem,
            ).wait()

    return _pos_broadcast_kernel


def kernel(feature, row_embed, col_embed):
    B, C, H, W = feature.shape
    half = C // 2
    out = pl.pallas_call(
        _make_pos_broadcast_kernel(B, H, W, half),
        in_specs=[
            pl.BlockSpec(memory_space=pltpu.MemorySpace.VMEM),
            pl.BlockSpec(memory_space=pltpu.MemorySpace.VMEM),
        ],
        out_specs=pl.BlockSpec(memory_space=pl.ANY),
        out_shape=jax.ShapeDtypeStruct((B, C, H * W), row_embed.dtype),
        scratch_shapes=[
            pltpu.VMEM((C, H * W), row_embed.dtype),
            pltpu.SemaphoreType.DMA,
        ],
    )(row_embed, col_embed)
    return out.reshape(B, C, H, W)
